# Initial kernel scaffold; baseline (speedup 1.0000x reference)
import jax
import jax.numpy as jnp
from jax.experimental import pallas as pl


def kernel(feature4, feature5, feature6, xyz, detect_point, W_ind, b_ind, W1, b1, W2, b2):
    raise NotImplementedError("write your pallas kernel here")


# speedup vs baseline: 16.7940x; 16.7940x over previous
import jax
import jax.numpy as jnp
from jax.experimental import pallas as pl
from jax.experimental.pallas import tpu as pltpu

KNN = 12


def _body(q_ref, xT_ref, f5_ref, Wi_ref, bi_ref, W1_ref, b1_ref, W2_ref,
          b2_ref, o_ref):
    q = q_ref[0]
    xT = xT_ref[0]
    TM = q.shape[0]
    N = xT.shape[1]

    qx = jnp.dot(q, xT, preferred_element_type=jnp.float32)
    q2 = jnp.sum(q * q, axis=1, keepdims=True)
    x2 = jnp.sum(xT * xT, axis=0, keepdims=True)
    d2 = q2 + x2 - 2.0 * qx

    BIG = jnp.float32(1e30)
    col = jax.lax.broadcasted_iota(jnp.int32, (TM, N), 1)

    work = d2
    wmat = jnp.zeros((TM, N), jnp.float32)
    wsum = jnp.zeros((TM, 1), jnp.float32)
    for _ in range(KNN):
        cur = jnp.min(work, axis=1, keepdims=True)
        pos = jnp.min(jnp.where(work == cur, col, N), axis=1,
                      keepdims=True)
        hit = col == pos
        w = 1.0 / (jnp.maximum(cur, 0.0) + 1e-8)
        wmat = jnp.where(hit, w, wmat)
        wsum = wsum + w
        work = jnp.where(hit, BIG, work)

    wn = wmat * (1.0 / wsum)
    interp = jnp.dot(wn, f5_ref[0], preferred_element_type=jnp.float32)
    h0 = jnp.maximum(jnp.dot(interp, Wi_ref[...],
                             preferred_element_type=jnp.float32)
                     + bi_ref[...], 0.0)
    h1 = jnp.maximum(jnp.dot(h0, W1_ref[...],
                             preferred_element_type=jnp.float32)
                     + b1_ref[...], 0.0)
    o_ref[0] = jnp.tanh(jnp.dot(h1, W2_ref[...],
                                preferred_element_type=jnp.float32)
                        + b2_ref[...])


def kernel(feature4, feature5, feature6, xyz, detect_point, W_ind, b_ind,
           W1, b1, W2, b2):
    B, M, _ = detect_point.shape
    _, N, C = feature5.shape
    H = W1.shape[1]
    TM = min(256, M)

    qpad = jnp.pad(detect_point, ((0, 0), (0, 0), (0, 5)))
    xT = jnp.pad(xyz, ((0, 0), (0, 0), (0, 5))).transpose(0, 2, 1)
    W2p = jnp.pad(W2, ((0, 0), (0, 5)))
    b2p = jnp.pad(b2, ((0, 5),)).reshape(1, 8)
    bi2 = b_ind.reshape(1, -1)
    b12 = b1.reshape(1, -1)

    out = pl.pallas_call(
        _body,
        grid=(B, M // TM),
        in_specs=[
            pl.BlockSpec((1, TM, 8), lambda b, m: (b, m, 0)),
            pl.BlockSpec((1, 8, N), lambda b, m: (b, 0, 0)),
            pl.BlockSpec((1, N, C), lambda b, m: (b, 0, 0)),
            pl.BlockSpec((C, C), lambda b, m: (0, 0)),
            pl.BlockSpec((1, C), lambda b, m: (0, 0)),
            pl.BlockSpec((C, H), lambda b, m: (0, 0)),
            pl.BlockSpec((1, H), lambda b, m: (0, 0)),
            pl.BlockSpec((H, 8), lambda b, m: (0, 0)),
            pl.BlockSpec((1, 8), lambda b, m: (0, 0)),
        ],
        out_specs=pl.BlockSpec((1, TM, 8), lambda b, m: (b, m, 0)),
        out_shape=jax.ShapeDtypeStruct((B, M, 8), jnp.float32),
        compiler_params=pltpu.CompilerParams(
            dimension_semantics=("parallel", "parallel")),
    )(qpad, xT, feature5, W_ind, bi2, W1, b12, W2p, b2p)
    return out[:, :, :3]


# speedup vs baseline: 24.5802x; 1.4636x over previous
import jax
import jax.numpy as jnp
from jax.experimental import pallas as pl
from jax.experimental.pallas import tpu as pltpu

KNN = 12


def _body(q_ref, xT_ref, f5_ref, Wi_ref, bi_ref, W1_ref, b1_ref, W2_ref,
          b2_ref, o_ref):
    q = q_ref[0]
    xT = xT_ref[0]
    TM = q.shape[0]
    N = xT.shape[1]

    qx = jnp.dot(q, xT, preferred_element_type=jnp.float32)
    q2 = jnp.sum(q * q, axis=1, keepdims=True)
    x2 = jnp.sum(xT * xT, axis=0, keepdims=True)
    d2 = jnp.maximum(q2 + x2 - 2.0 * qx, 0.0)

    col = jax.lax.broadcasted_iota(jnp.int32, (TM, N), 1)
    key = jax.lax.bitcast_convert_type(d2, jnp.int32)
    key = (key & jnp.int32(~(N - 1))) | col

    INTMAX = jnp.int32(0x7FFFFFFF)
    t = jnp.full((TM, 1), -1, jnp.int32)
    for _ in range(KNN):
        t = jnp.min(jnp.where(key <= t, INTMAX, key), axis=1, keepdims=True)

    wraw = 1.0 / (d2 + 1e-8)
    wmat = jnp.where(key <= t, wraw, 0.0)
    wsum = jnp.sum(wmat, axis=1, keepdims=True)
    wn = wmat * (1.0 / wsum)

    interp = jnp.dot(wn, f5_ref[0], preferred_element_type=jnp.float32)
    h0 = jnp.maximum(jnp.dot(interp, Wi_ref[...],
                             preferred_element_type=jnp.float32)
                     + bi_ref[...], 0.0)
    h1 = jnp.maximum(jnp.dot(h0, W1_ref[...],
                             preferred_element_type=jnp.float32)
                     + b1_ref[...], 0.0)
    o_ref[0] = jnp.tanh(jnp.dot(h1, W2_ref[...],
                                preferred_element_type=jnp.float32)
                        + b2_ref[...])


def kernel(feature4, feature5, feature6, xyz, detect_point, W_ind, b_ind,
           W1, b1, W2, b2):
    B, M, _ = detect_point.shape
    _, N, C = feature5.shape
    H = W1.shape[1]
    TM = min(256, M)

    qpad = jnp.pad(detect_point, ((0, 0), (0, 0), (0, 5)))
    xT = jnp.pad(xyz, ((0, 0), (0, 0), (0, 5))).transpose(0, 2, 1)
    W2p = jnp.pad(W2, ((0, 0), (0, 5)))
    b2p = jnp.pad(b2, ((0, 5),)).reshape(1, 8)
    bi2 = b_ind.reshape(1, -1)
    b12 = b1.reshape(1, -1)

    out = pl.pallas_call(
        _body,
        grid=(B, M // TM),
        in_specs=[
            pl.BlockSpec((1, TM, 8), lambda b, m: (b, m, 0)),
            pl.BlockSpec((1, 8, N), lambda b, m: (b, 0, 0)),
            pl.BlockSpec((1, N, C), lambda b, m: (b, 0, 0)),
            pl.BlockSpec((C, C), lambda b, m: (0, 0)),
            pl.BlockSpec((1, C), lambda b, m: (0, 0)),
            pl.BlockSpec((C, H), lambda b, m: (0, 0)),
            pl.BlockSpec((1, H), lambda b, m: (0, 0)),
            pl.BlockSpec((H, 8), lambda b, m: (0, 0)),
            pl.BlockSpec((1, 8), lambda b, m: (0, 0)),
        ],
        out_specs=pl.BlockSpec((1, TM, 8), lambda b, m: (b, m, 0)),
        out_shape=jax.ShapeDtypeStruct((B, M, 8), jnp.float32),
        compiler_params=pltpu.CompilerParams(
            dimension_semantics=("parallel", "parallel")),
    )(qpad, xT, feature5, W_ind, bi2, W1, b12, W2p, b2p)
    return out[:, :, :3]


# speedup vs baseline: 27.5754x; 1.1219x over previous
import jax
import jax.numpy as jnp
from jax.experimental import pallas as pl
from jax.experimental.pallas import tpu as pltpu

KNN = 12


def _body(q_ref, xT_ref, f5_ref, Wi_ref, bi_ref, W1_ref, b1_ref, W2_ref,
          b2_ref, o_ref):
    q = q_ref[0]
    xT = xT_ref[0]
    TM = q.shape[0]
    N = xT.shape[1]

    qx = jnp.dot(q, xT, preferred_element_type=jnp.float32)
    q2 = jnp.sum(q * q, axis=1, keepdims=True)
    x2 = jnp.sum(xT * xT, axis=0, keepdims=True)
    d2 = jnp.maximum(q2 + x2 - 2.0 * qx, 0.0)

    SIGN = jnp.int32(-2**31)
    colx = jax.lax.broadcasted_iota(jnp.int32, (TM, N), 1) | SIGN
    key = (jax.lax.bitcast_convert_type(d2, jnp.int32) & jnp.int32(~(N - 1))
           ) | colx

    off = jnp.zeros((TM, 1), jnp.int32)
    for _ in range(KNN):
        m = jnp.min(key - off, axis=1, keepdims=True)
        off = off + m + jnp.int32(-2**31 + 1)
    t = off + jnp.int32(2**31 - 1)

    wraw = 1.0 / (d2 + 1e-8)
    wmat = jnp.where(key <= t, wraw, 0.0)
    wsum = jnp.sum(wmat, axis=1, keepdims=True)
    wn = wmat * (1.0 / wsum)

    interp = jnp.dot(wn, f5_ref[0], preferred_element_type=jnp.float32)
    h0 = jnp.maximum(jnp.dot(interp, Wi_ref[...],
                             preferred_element_type=jnp.float32)
                     + bi_ref[...], 0.0)
    h1 = jnp.maximum(jnp.dot(h0, W1_ref[...],
                             preferred_element_type=jnp.float32)
                     + b1_ref[...], 0.0)
    o_ref[0] = jnp.tanh(jnp.dot(h1, W2_ref[...],
                                preferred_element_type=jnp.float32)
                        + b2_ref[...])


def kernel(feature4, feature5, feature6, xyz, detect_point, W_ind, b_ind,
           W1, b1, W2, b2):
    B, M, _ = detect_point.shape
    _, N, C = feature5.shape
    H = W1.shape[1]
    TM = min(256, M)

    qpad = jnp.pad(detect_point, ((0, 0), (0, 0), (0, 5)))
    xT = jnp.pad(xyz, ((0, 0), (0, 0), (0, 5))).transpose(0, 2, 1)
    W2p = jnp.pad(W2, ((0, 0), (0, 5)))
    b2p = jnp.pad(b2, ((0, 5),)).reshape(1, 8)
    bi2 = b_ind.reshape(1, -1)
    b12 = b1.reshape(1, -1)

    out = pl.pallas_call(
        _body,
        grid=(B, M // TM),
        in_specs=[
            pl.BlockSpec((1, TM, 8), lambda b, m: (b, m, 0)),
            pl.BlockSpec((1, 8, N), lambda b, m: (b, 0, 0)),
            pl.BlockSpec((1, N, C), lambda b, m: (b, 0, 0)),
            pl.BlockSpec((C, C), lambda b, m: (0, 0)),
            pl.BlockSpec((1, C), lambda b, m: (0, 0)),
            pl.BlockSpec((C, H), lambda b, m: (0, 0)),
            pl.BlockSpec((1, H), lambda b, m: (0, 0)),
            pl.BlockSpec((H, 8), lambda b, m: (0, 0)),
            pl.BlockSpec((1, 8), lambda b, m: (0, 0)),
        ],
        out_specs=pl.BlockSpec((1, TM, 8), lambda b, m: (b, m, 0)),
        out_shape=jax.ShapeDtypeStruct((B, M, 8), jnp.float32),
        compiler_params=pltpu.CompilerParams(
            dimension_semantics=("parallel", "parallel")),
    )(qpad, xT, feature5, W_ind, bi2, W1, b12, W2p, b2p)
    return out[:, :, :3]


# speedup vs baseline: 29.3301x; 1.0636x over previous
import jax
import jax.numpy as jnp
from jax.experimental import pallas as pl
from jax.experimental.pallas import tpu as pltpu

KNN = 12


def _body(q_ref, xT_ref, f5_ref, Wi_ref, bi_ref, W1_ref, b1_ref, W2_ref,
          b2_ref, o_ref):
    q = q_ref[0]
    xT = xT_ref[0]
    TM = q.shape[0]
    N = xT.shape[1]

    qx = jnp.dot(q, xT, preferred_element_type=jnp.float32)
    q2 = jnp.sum(q * q, axis=1, keepdims=True)
    x2 = jnp.sum(xT * xT, axis=0, keepdims=True)
    d2 = jnp.maximum(q2 + x2 - 2.0 * qx, 0.0)

    SIGN = jnp.int32(-2**31)
    colx = jax.lax.broadcasted_iota(jnp.int32, (TM, N), 1) | SIGN
    key = (jax.lax.bitcast_convert_type(d2, jnp.int32) & jnp.int32(~(N - 1))
           ) | colx

    off = jnp.zeros((TM, 1), jnp.int32)
    for _ in range(KNN):
        m = jnp.min(key - off, axis=1, keepdims=True)
        off = off + m + jnp.int32(-2**31 + 1)
    t = off + jnp.int32(2**31 - 1)

    wraw = 1.0 / (d2 + 1e-8)
    wmat = jnp.where(key <= t, wraw, 0.0)
    wsum = jnp.sum(wmat, axis=1, keepdims=True)
    wn = (wmat * (1.0 / wsum)).astype(jnp.bfloat16)

    interp = jnp.dot(wn, f5_ref[0], preferred_element_type=jnp.float32)
    h0 = jnp.maximum(jnp.dot(interp.astype(jnp.bfloat16), Wi_ref[...],
                             preferred_element_type=jnp.float32)
                     + bi_ref[...], 0.0)
    h1 = jnp.maximum(jnp.dot(h0.astype(jnp.bfloat16), W1_ref[...],
                             preferred_element_type=jnp.float32)
                     + b1_ref[...], 0.0)
    o_ref[0] = jnp.tanh(jnp.dot(h1, W2_ref[...],
                                preferred_element_type=jnp.float32)
                        + b2_ref[...])


def kernel(feature4, feature5, feature6, xyz, detect_point, W_ind, b_ind,
           W1, b1, W2, b2):
    B, M, _ = detect_point.shape
    _, N, C = feature5.shape
    H = W1.shape[1]
    TM = min(256, M)

    qpad = jnp.pad(detect_point, ((0, 0), (0, 0), (0, 5)))
    xT = jnp.pad(xyz, ((0, 0), (0, 0), (0, 5))).transpose(0, 2, 1)
    f5h = feature5.astype(jnp.bfloat16)
    Wih = W_ind.astype(jnp.bfloat16)
    W1h = W1.astype(jnp.bfloat16)
    W2p = jnp.pad(W2, ((0, 0), (0, 5)))
    b2p = jnp.pad(b2, ((0, 5),)).reshape(1, 8)
    bi2 = b_ind.reshape(1, -1)
    b12 = b1.reshape(1, -1)

    out = pl.pallas_call(
        _body,
        grid=(B, M // TM),
        in_specs=[
            pl.BlockSpec((1, TM, 8), lambda b, m: (b, m, 0)),
            pl.BlockSpec((1, 8, N), lambda b, m: (b, 0, 0)),
            pl.BlockSpec((1, N, C), lambda b, m: (b, 0, 0)),
            pl.BlockSpec((C, C), lambda b, m: (0, 0)),
            pl.BlockSpec((1, C), lambda b, m: (0, 0)),
            pl.BlockSpec((C, H), lambda b, m: (0, 0)),
            pl.BlockSpec((1, H), lambda b, m: (0, 0)),
            pl.BlockSpec((H, 8), lambda b, m: (0, 0)),
            pl.BlockSpec((1, 8), lambda b, m: (0, 0)),
        ],
        out_specs=pl.BlockSpec((1, TM, 8), lambda b, m: (b, m, 0)),
        out_shape=jax.ShapeDtypeStruct((B, M, 8), jnp.float32),
        compiler_params=pltpu.CompilerParams(
            dimension_semantics=("parallel", "parallel")),
    )(qpad, xT, f5h, Wih, bi2, W1h, b12, W2p, b2p)
    return out[:, :, :3]
